# baseline (speedup 1.0000x reference)
import functools

import jax
import jax.numpy as jnp
from jax.experimental import pallas as pl
from jax.experimental.pallas import tpu as pltpu

_TOTAL = 16384
_D = 512
_NB = 16
_BLK = 8192


def _fused_body(x_ref, w_ref, benc_ref, wagg_ref, starts_ref, ends_ref,
                bagg_ref, out_ref, acc_ref):
    i = pl.program_id(0)

    @pl.when(i == 0)
    def _init():
        acc_ref[...] = jnp.zeros_like(acc_ref)

    h = jnp.maximum(
        jnp.dot(x_ref[...], w_ref[...], preferred_element_type=jnp.float32)
        + benc_ref[...], 0.0)
    s = jnp.sum(h * wagg_ref[...], axis=1, keepdims=True)

    rows = i * _BLK + jax.lax.broadcasted_iota(jnp.int32, (_NB, _BLK), 1)
    mask = ((rows >= starts_ref[...]) & (rows < ends_ref[...])
            ).astype(jnp.float32)
    acc_ref[...] += jnp.dot(mask, s, preferred_element_type=jnp.float32)

    @pl.when(i == pl.num_programs(0) - 1)
    def _fin():
        counts = (ends_ref[...] - starts_ref[...]).astype(jnp.float32)
        counts = jnp.maximum(counts, 1.0)
        out_ref[...] = acc_ref[...] / counts + bagg_ref[...]


def kernel(x, bag_sizes, W_enc, b_enc, W_agg, b_agg):
    starts = bag_sizes[:_NB].reshape(_NB, 1)
    ends = bag_sizes[1:].reshape(_NB, 1)
    benc = b_enc.reshape(1, _D)
    bagg = b_agg.reshape(1, 1)

    grid = _TOTAL // _BLK
    out = pl.pallas_call(
        _fused_body,
        grid=(grid,),
        in_specs=[
            pl.BlockSpec((_BLK, _D), lambda i: (i, 0)),
            pl.BlockSpec((_D, _D), lambda i: (0, 0)),
            pl.BlockSpec((1, _D), lambda i: (0, 0)),
            pl.BlockSpec((1, _D), lambda i: (0, 0)),
            pl.BlockSpec((_NB, 1), lambda i: (0, 0)),
            pl.BlockSpec((_NB, 1), lambda i: (0, 0)),
            pl.BlockSpec((1, 1), lambda i: (0, 0)),
        ],
        out_specs=pl.BlockSpec((_NB, 1), lambda i: (0, 0)),
        out_shape=jax.ShapeDtypeStruct((_NB, 1), jnp.float32),
        scratch_shapes=[pltpu.VMEM((_NB, 1), jnp.float32)],
        compiler_params=pltpu.CompilerParams(
            dimension_semantics=("arbitrary",)),
    )(x, W_enc, benc, W_agg.reshape(1, _D), starts, ends, bagg)
    return out


# speedup vs baseline: 1.0624x; 1.0624x over previous
import functools

import jax
import jax.numpy as jnp
from jax.experimental import pallas as pl
from jax.experimental.pallas import tpu as pltpu

_TOTAL = 16384
_D = 512
_NB = 16
_BLK = 4096


def _fused_body(x_ref, w_ref, benc_ref, wagg_ref, starts_ref, ends_ref,
                bagg_ref, out_ref, acc_ref):
    i = pl.program_id(0)

    @pl.when(i == 0)
    def _init():
        acc_ref[...] = jnp.zeros_like(acc_ref)

    h = jnp.maximum(
        jnp.dot(x_ref[...], w_ref[...], preferred_element_type=jnp.float32)
        + benc_ref[...], 0.0)
    s = jnp.sum(h * wagg_ref[...], axis=1, keepdims=True)

    rows = i * _BLK + jax.lax.broadcasted_iota(jnp.int32, (_NB, _BLK), 1)
    mask = ((rows >= starts_ref[...]) & (rows < ends_ref[...])
            ).astype(jnp.float32)
    acc_ref[...] += jnp.dot(mask, s, preferred_element_type=jnp.float32)

    @pl.when(i == pl.num_programs(0) - 1)
    def _fin():
        counts = (ends_ref[...] - starts_ref[...]).astype(jnp.float32)
        counts = jnp.maximum(counts, 1.0)
        out_ref[...] = acc_ref[...] / counts + bagg_ref[...]


def kernel(x, bag_sizes, W_enc, b_enc, W_agg, b_agg):
    starts = bag_sizes[:_NB].reshape(_NB, 1)
    ends = bag_sizes[1:].reshape(_NB, 1)
    benc = b_enc.reshape(1, _D)
    bagg = b_agg.reshape(1, 1)

    grid = _TOTAL // _BLK
    out = pl.pallas_call(
        _fused_body,
        grid=(grid,),
        in_specs=[
            pl.BlockSpec((_BLK, _D), lambda i: (i, 0)),
            pl.BlockSpec((_D, _D), lambda i: (0, 0)),
            pl.BlockSpec((1, _D), lambda i: (0, 0)),
            pl.BlockSpec((1, _D), lambda i: (0, 0)),
            pl.BlockSpec((_NB, 1), lambda i: (0, 0)),
            pl.BlockSpec((_NB, 1), lambda i: (0, 0)),
            pl.BlockSpec((1, 1), lambda i: (0, 0)),
        ],
        out_specs=pl.BlockSpec((_NB, 1), lambda i: (0, 0)),
        out_shape=jax.ShapeDtypeStruct((_NB, 1), jnp.float32),
        scratch_shapes=[pltpu.VMEM((_NB, 1), jnp.float32)],
        compiler_params=pltpu.CompilerParams(
            dimension_semantics=("arbitrary",)),
    )(x, W_enc, benc, W_agg.reshape(1, _D), starts, ends, bagg)
    return out
